# baseline (speedup 1.0000x reference)
import jax
import jax.numpy as jnp
from jax.experimental import pallas as pl
from jax.experimental.pallas import tpu as pltpu

B = 128
H = 64
CS = 100
R = 4


def _body(users_ref, items_ref, um_any, im_any,
          wih_u, whh_u, bih_u, bhh_u, wih_i, whh_i, bih_i, bhh_i,
          urow, irow,
          ue_out, ie_out, umo_any, imo_any,
          slab_u, slab_i,
          ibuf_u, ibuf_i, obuf_u, obuf_i,
          sem_iu, sem_ii, sem_ou, sem_oi, sem_g):
    n = um_any.shape[0]
    t = n // CS

    def in_u(k, slot):
        return pltpu.make_async_copy(um_any.at[pl.ds(k * CS, CS)],
                                     ibuf_u.at[slot], sem_iu.at[slot])

    def in_i(k, slot):
        return pltpu.make_async_copy(im_any.at[pl.ds(k * CS, CS)],
                                     ibuf_i.at[slot], sem_ii.at[slot])

    def out_u(k, slot):
        return pltpu.make_async_copy(obuf_u.at[slot],
                                     umo_any.at[pl.ds(k * CS, CS)], sem_ou.at[slot])

    def out_i(k, slot):
        return pltpu.make_async_copy(obuf_i.at[slot],
                                     imo_any.at[pl.ds(k * CS, CS)], sem_oi.at[slot])

    for b in range(B):
        pltpu.make_async_copy(um_any.at[users_ref[b]], slab_u.at[b], sem_g).start()
        pltpu.make_async_copy(im_any.at[items_ref[b]], slab_i.at[b], sem_g).start()
    for r0 in range(R):
        in_u(r0, r0).start()
        in_i(r0, r0).start()
    for b in range(B):
        pltpu.make_async_copy(um_any.at[users_ref[b]], slab_u.at[b], sem_g).wait()
        pltpu.make_async_copy(im_any.at[items_ref[b]], slab_i.at[b], sem_g).wait()

    eq3 = (jax.lax.broadcasted_iota(jnp.int32, (B, H, B), 0)
           == jax.lax.broadcasted_iota(jnp.int32, (B, H, B), 2))
    ueT = jnp.sum(jnp.where(eq3, slab_u[...], 0.0), axis=0)
    ieT = jnp.sum(jnp.where(eq3, slab_i[...], 0.0), axis=0)

    def gru_t(xT, hT, wih, whh, bih, bhh):
        giT = jnp.dot(wih, xT, preferred_element_type=jnp.float32) + bih
        ghT = jnp.dot(whh, hT, preferred_element_type=jnp.float32) + bhh
        r = jax.nn.sigmoid(giT[:H] + ghT[:H])
        z = jax.nn.sigmoid(giT[H:2 * H] + ghT[H:2 * H])
        nn = jnp.tanh(giT[2 * H:] + r * ghT[2 * H:])
        return (1.0 - z) * nn + z * hT

    xT_u = jnp.concatenate([ueT, ieT], axis=0)
    xT_i = jnp.concatenate([ieT, ueT], axis=0)
    nu = gru_t(xT_u, ueT, wih_u[...], whh_u[...], bih_u[...], bhh_u[...])
    ni = gru_t(xT_i, ieT, wih_i[...], whh_i[...], bih_i[...], bhh_i[...])
    ue_out[...] = nu
    ie_out[...] = ni

    iota0 = jax.lax.broadcasted_iota(jnp.int32, (CS, H, B), 0)

    def step(k, carry):
        slot = jax.lax.rem(k, R)
        in_u(k, slot).wait()
        in_i(k, slot).wait()

        @pl.when(k >= R)
        def _():
            out_u(k - R, slot).wait()
            out_i(k - R, slot).wait()

        locs_u = (urow[...] - k * CS)[None]
        locs_i = (irow[...] - k * CS)[None]
        obuf_u.at[slot][...] = jnp.where(iota0 == locs_u, nu[None],
                                         ibuf_u.at[slot][...])
        obuf_i.at[slot][...] = jnp.where(iota0 == locs_i, ni[None],
                                         ibuf_i.at[slot][...])
        out_u(k, slot).start()
        out_i(k, slot).start()

        @pl.when(k + R < t)
        def _():
            in_u(k + R, slot).start()
            in_i(k + R, slot).start()
        return carry

    jax.lax.fori_loop(0, t, step, 0)

    for r0 in range(R):
        k = t - R + r0
        out_u(k, k % R).wait()
        out_i(k, k % R).wait()


def kernel(user_memory, item_memory, users, items,
           W_ih_u, W_hh_u, b_ih_u, b_hh_u,
           W_ih_i, W_hh_i, b_ih_i, b_hh_i):
    users = users.astype(jnp.int32)
    items = items.astype(jnp.int32)
    um_t = jnp.transpose(user_memory, (1, 2, 0))
    im_t = jnp.transpose(item_memory, (1, 2, 0))

    out_shape = (
        jax.ShapeDtypeStruct((H, B), jnp.float32),
        jax.ShapeDtypeStruct((H, B), jnp.float32),
        jax.ShapeDtypeStruct(um_t.shape, jnp.float32),
        jax.ShapeDtypeStruct(im_t.shape, jnp.float32),
    )
    full = pl.BlockSpec(memory_space=pltpu.VMEM)
    anyspec = pl.BlockSpec(memory_space=pl.ANY)
    grid_spec = pltpu.PrefetchScalarGridSpec(
        num_scalar_prefetch=2,
        grid=(),
        in_specs=[anyspec, anyspec] + [full] * 10,
        out_specs=[full, full, anyspec, anyspec],
        scratch_shapes=[
            pltpu.VMEM((B, H, B), jnp.float32),
            pltpu.VMEM((B, H, B), jnp.float32),
            pltpu.VMEM((R, CS, H, B), jnp.float32),
            pltpu.VMEM((R, CS, H, B), jnp.float32),
            pltpu.VMEM((R, CS, H, B), jnp.float32),
            pltpu.VMEM((R, CS, H, B), jnp.float32),
            pltpu.SemaphoreType.DMA((R,)),
            pltpu.SemaphoreType.DMA((R,)),
            pltpu.SemaphoreType.DMA((R,)),
            pltpu.SemaphoreType.DMA((R,)),
            pltpu.SemaphoreType.DMA,
        ],
    )
    ueT, ieT, new_um_t, new_im_t = pl.pallas_call(
        _body,
        grid_spec=grid_spec,
        out_shape=out_shape,
        compiler_params=pltpu.CompilerParams(
            vmem_limit_bytes=110 * 1024 * 1024),
        name="limnet_step",
    )(users, items, um_t, im_t,
      W_ih_u, W_hh_u, b_ih_u.reshape(3 * H, 1), b_hh_u.reshape(3 * H, 1),
      W_ih_i, W_hh_i, b_ih_i.reshape(3 * H, 1), b_hh_i.reshape(3 * H, 1),
      users.reshape(1, B), items.reshape(1, B))
    new_um = jnp.transpose(new_um_t, (2, 0, 1))
    new_im = jnp.transpose(new_im_t, (2, 0, 1))
    return (ueT.T, ieT.T, new_um, new_im)
